# baseline (speedup 1.0000x reference)
import jax
import jax.numpy as jnp
from jax.experimental import pallas as pl
from jax.experimental.pallas import tpu as pltpu

_CHUNK = 512
_NORM_PERSISTENT = 1024.0


def _qkproj_kernel(q_ref, k_ref, mp_ref, out_ref, m_acc, mb_acc, n_acc):
    i = pl.program_id(0)

    @pl.when(i == 0)
    def _init():
        mp = mp_ref[...]
        m_acc[...] = mp
        mb_acc[...] = mp.astype(jnp.bfloat16)
        n_acc[0, 0] = _NORM_PERSISTENT

    q = q_ref[...]
    k = k_ref[...]
    qb = q.astype(jnp.bfloat16)
    kb = k.astype(jnp.bfloat16)

    row = jax.lax.broadcasted_iota(jnp.int32, (_CHUNK, _CHUNK), 0)
    col = jax.lax.broadcasted_iota(jnp.int32, (_CHUNK, _CHUNK), 1)
    causal = (col <= row)

    ss = jnp.sum(k * k, axis=1, keepdims=True)
    csum = jnp.dot(causal.astype(jnp.float32), ss,
                   preferred_element_type=jnp.float32)
    norms = n_acc[0, 0] + csum
    n_acc[0, 0] = n_acc[0, 0] + jnp.sum(ss)

    scores = jax.lax.dot_general(qb, kb, (((1,), (1,)), ((), ())),
                                 preferred_element_type=jnp.float32)
    scores = jnp.where(causal, scores, 0.0).astype(jnp.bfloat16)

    out = jax.lax.dot_general(qb, mb_acc[...], (((1,), (1,)), ((), ())),
                              preferred_element_type=jnp.float32)
    out = out + jax.lax.dot_general(scores, kb, (((1,), (0,)), ((), ())),
                                    preferred_element_type=jnp.float32)
    out_ref[...] = out / jnp.maximum(norms, 1e-8)

    m_new = m_acc[...] + jax.lax.dot_general(kb, kb, (((0,), (0,)), ((), ())),
                                             preferred_element_type=jnp.float32)
    m_acc[...] = m_new
    mb_acc[...] = m_new.astype(jnp.bfloat16)


def kernel(queries, keys, m_persistent):
    t_len, dim = queries.shape
    n_chunks = t_len // _CHUNK
    return pl.pallas_call(
        _qkproj_kernel,
        out_shape=jax.ShapeDtypeStruct((t_len, dim), jnp.float32),
        grid=(n_chunks,),
        in_specs=[
            pl.BlockSpec((_CHUNK, dim), lambda i: (i, 0)),
            pl.BlockSpec((_CHUNK, dim), lambda i: (i, 0)),
            pl.BlockSpec((dim, dim), lambda i: (0, 0)),
        ],
        out_specs=pl.BlockSpec((_CHUNK, dim), lambda i: (i, 0)),
        scratch_shapes=[
            pltpu.VMEM((dim, dim), jnp.float32),
            pltpu.VMEM((dim, dim), jnp.bfloat16),
            pltpu.SMEM((1, 1), jnp.float32),
        ],
        compiler_params=pltpu.CompilerParams(
            dimension_semantics=("arbitrary",),
        ),
        name="qkprojection",
    )(queries, keys, m_persistent)
